# baseline (speedup 1.0000x reference)
import functools

import jax
import jax.numpy as jnp
from jax import lax
from jax.experimental import pallas as pl
from jax.experimental.pallas import tpu as pltpu
from jax.experimental.pallas import tpu_sc as plsc

_N = 10000
_E = 320000
_D = 128
_ALPHA = 0.5

_NTILES = 16
_EPT = _E // _NTILES
_CHUNK = 125
_NCHUNK = _EPT // _CHUNK
_SB = 8
_NBLK = _NCHUNK // _SB
_NPAD = 10240
_ZROWS = _NPAD // _NTILES

def _degree_body(ei_hbm, zeros_hbm, ones_hbm, deg_hbm, idx_v, ones_v, acc):
    c = lax.axis_index("c")
    s = lax.axis_index("s")
    pltpu.sync_copy(zeros_hbm.at[pl.ds(0, _ZROWS)], acc.at[pl.ds(s * _ZROWS, _ZROWS)])
    pltpu.sync_copy(ones_hbm, ones_v)
    pltpu.sync_copy(ei_hbm.at[c, s], idx_v)
    plsc.subcore_barrier()

    def body(j, carry):
        pltpu.sync_copy(ones_v, acc.at[idx_v.at[j]], add=True)
        return carry

    lax.fori_loop(0, _NCHUNK, body, 0)
    plsc.subcore_barrier()
    pltpu.sync_copy(acc.at[pl.ds(s * _ZROWS, _ZROWS)],
                    deg_hbm.at[c, pl.ds(s * _ZROWS, _ZROWS)])


def _proj_body(x_ref, wsrc_ref, wdst_ref, deg_ref, zsrc_ref, zdst_ref):
    i = pl.program_id(0)
    bm = x_ref.shape[0]
    xb = x_ref[...]
    dout = deg_ref[0, pl.ds(i * bm, bm)]
    din = deg_ref[1, pl.ds(i * bm, bm)]
    so = jnp.where(dout > 0, lax.rsqrt(dout), 0.0)
    si = jnp.where(din > 0, lax.rsqrt(din), 0.0)
    dims = (((1,), (1,)), ((), ()))
    ysrc = lax.dot_general(xb, wsrc_ref[...], dims, preferred_element_type=jnp.float32)
    ydst = lax.dot_general(xb, wdst_ref[...], dims, preferred_element_type=jnp.float32)
    zsrc_ref[...] = (_ALPHA * ysrc) * si[:, None]
    zdst_ref[...] = ((1.0 - _ALPHA) * ydst) * so[:, None]


def _project(x, w_src, w_dst, deg):
    bm = 2048
    grid = (pl.cdiv(_N, bm),)
    return pl.pallas_call(
        _proj_body,
        grid=grid,
        in_specs=[
            pl.BlockSpec((bm, _D), lambda i: (i, 0)),
            pl.BlockSpec((_D, _D), lambda i: (0, 0)),
            pl.BlockSpec((_D, _D), lambda i: (0, 0)),
            pl.BlockSpec((2, _NPAD), lambda i: (0, 0)),
        ],
        out_specs=[
            pl.BlockSpec((bm, _D), lambda i: (i, 0)),
            pl.BlockSpec((bm, _D), lambda i: (i, 0)),
        ],
        out_shape=[
            jax.ShapeDtypeStruct((_N, _D), jnp.float32),
            jax.ShapeDtypeStruct((_N, _D), jnp.float32),
        ],
    )(x, w_src, w_dst, deg)


def _agg_body(ei_hbm, zsrc_hbm, zdst_hbm, zeros_hbm, sf_hbm, sb_hbm,
              gidx_v, sidx_v, rows_v, acc, gsem, ssem):
    c = lax.axis_index("c")
    s = lax.axis_index("s")

    def run(table_hbm, out_hbm, gdir, sdir):
        pltpu.sync_copy(zeros_hbm, acc.at[pl.ds(s * _ZROWS, _ZROWS), :])
        pltpu.sync_copy(ei_hbm.at[gdir, s, pl.ds(0, _SB)], gidx_v.at[0])
        pltpu.sync_copy(ei_hbm.at[sdir, s, pl.ds(0, _SB)], sidx_v.at[0])
        plsc.subcore_barrier()
        pltpu.async_copy(table_hbm.at[gidx_v.at[0, 0]], rows_v.at[0], gsem.at[0])

        def body(j, carry):
            buf = lax.rem(j, 2)
            blk = lax.div(j, _SB)
            slot = lax.rem(blk, 2)
            off = lax.rem(j, _SB)
            pltpu.make_async_copy(
                table_hbm.at[gidx_v.at[slot, off]], rows_v.at[buf], gsem.at[buf]
            ).wait()
            pltpu.async_copy(rows_v.at[buf], acc.at[sidx_v.at[slot, off]],
                             ssem.at[buf], add=True)

            @pl.when(j >= 1)
            def _():
                jp = j - 1
                pltpu.make_async_copy(
                    rows_v.at[lax.rem(jp, 2)],
                    acc.at[sidx_v.at[lax.rem(lax.div(jp, _SB), 2), lax.rem(jp, _SB)]],
                    ssem.at[lax.rem(jp, 2)],
                ).wait()

            @pl.when(jnp.logical_and(off == 0, blk + 1 < _NBLK))
            def _():
                nslot = lax.rem(blk + 1, 2)
                pltpu.sync_copy(ei_hbm.at[gdir, s, pl.ds((blk + 1) * _SB, _SB)], gidx_v.at[nslot])
                pltpu.sync_copy(ei_hbm.at[sdir, s, pl.ds((blk + 1) * _SB, _SB)], sidx_v.at[nslot])

            jn = j + 1

            @pl.when(jn < _NCHUNK)
            def _():
                bufn = lax.rem(jn, 2)
                pltpu.async_copy(
                    table_hbm.at[gidx_v.at[lax.rem(lax.div(jn, _SB), 2), lax.rem(jn, _SB)]],
                    rows_v.at[bufn], gsem.at[bufn])

            return carry

        lax.fori_loop(0, _NCHUNK, body, 0)
        _last = _NCHUNK - 1
        pltpu.make_async_copy(
            rows_v.at[_last % 2],
            acc.at[sidx_v.at[(_last // _SB) % 2, _last % _SB]],
            ssem.at[_last % 2],
        ).wait()
        plsc.subcore_barrier()
        pltpu.sync_copy(acc.at[pl.ds(s * _ZROWS, _ZROWS), :],
                        out_hbm.at[pl.ds(s * _ZROWS, _ZROWS), :])

    @pl.when(c == 0)
    def _():
        run(zsrc_hbm, sf_hbm, 1, 0)

    @pl.when(c == 1)
    def _():
        run(zdst_hbm, sb_hbm, 0, 1)


def _epi_body(sf_ref, sb_ref, deg_ref, bsrc_ref, bdst_ref, o_ref):
    i = pl.program_id(0)
    bm = sf_ref.shape[0]
    dout = deg_ref[0, pl.ds(i * bm, bm)]
    din = deg_ref[1, pl.ds(i * bm, bm)]
    so = jnp.where(dout > 0, lax.rsqrt(dout), 0.0)
    si = jnp.where(din > 0, lax.rsqrt(din), 0.0)
    bias = _ALPHA * bsrc_ref[...] + (1.0 - _ALPHA) * bdst_ref[...]
    o_ref[...] = sf_ref[...] * so[:, None] + sb_ref[...] * si[:, None] + bias


def _epilogue(sf, sb, deg, b_src, b_dst):
    bm = 2048
    grid = (pl.cdiv(_N, bm),)
    return pl.pallas_call(
        _epi_body,
        grid=grid,
        in_specs=[
            pl.BlockSpec((bm, _D), lambda i: (i, 0)),
            pl.BlockSpec((bm, _D), lambda i: (i, 0)),
            pl.BlockSpec((2, _NPAD), lambda i: (0, 0)),
            pl.BlockSpec((1, _D), lambda i: (0, 0)),
            pl.BlockSpec((1, _D), lambda i: (0, 0)),
        ],
        out_specs=pl.BlockSpec((bm, _D), lambda i: (i, 0)),
        out_shape=jax.ShapeDtypeStruct((_N, _D), jnp.float32),
    )(sf, sb, deg, b_src, b_dst)


@functools.cache
def _sc_kernels():
    mesh = plsc.VectorSubcoreMesh(core_axis_name="c", subcore_axis_name="s")
    degree_kernel = pl.kernel(
        _degree_body,
        mesh=mesh,
        out_type=jax.ShapeDtypeStruct((2, _NPAD), jnp.float32),
        scratch_types=[
            pltpu.VMEM((_NCHUNK, _CHUNK), jnp.int32),
            pltpu.VMEM((_CHUNK,), jnp.float32),
            pltpu.VMEM_SHARED((_NPAD,), jnp.float32),
        ],
    )
    agg_kernel = pl.kernel(
        _agg_body,
        mesh=mesh,
        out_type=[
            jax.ShapeDtypeStruct((_NPAD, _D), jnp.float32),
            jax.ShapeDtypeStruct((_NPAD, _D), jnp.float32),
        ],
        scratch_types=[
            pltpu.VMEM((2, _SB, _CHUNK), jnp.int32),
            pltpu.VMEM((2, _SB, _CHUNK), jnp.int32),
            pltpu.VMEM((2, _CHUNK, _D), jnp.float32),
            pltpu.VMEM_SHARED((_NPAD, _D), jnp.float32),
            pltpu.SemaphoreType.DMA((2,)),
            pltpu.SemaphoreType.DMA((2,)),
        ],
    )
    return degree_kernel, agg_kernel


def kernel(x, edge_index, W_src, b_src, W_dst, b_dst):
    degree_kernel, agg_kernel = _sc_kernels()
    ei = edge_index.astype(jnp.int32).reshape(2, _NTILES, _NCHUNK, _CHUNK)
    zeros1 = jnp.zeros((_NPAD,), jnp.float32)
    ones = jnp.ones((_CHUNK,), jnp.float32)
    zeros2 = jnp.zeros((_ZROWS, _D), jnp.float32)

    deg = degree_kernel(ei, zeros1, ones)
    z_src, z_dst = _project(x, W_src, W_dst, deg)
    s_fwd, s_bwd = agg_kernel(ei, z_src, z_dst, zeros2)
    return _epilogue(s_fwd, s_bwd, deg,
                     b_src.reshape(1, _D), b_dst.reshape(1, _D))


# speedup vs baseline: 1.0460x; 1.0460x over previous
import functools

import jax
import jax.numpy as jnp
from jax import lax
from jax.experimental import pallas as pl
from jax.experimental.pallas import tpu as pltpu
from jax.experimental.pallas import tpu_sc as plsc

_N = 10000
_E = 320000
_D = 128
_ALPHA = 0.5

_NTILES = 16
_EPT = _E // _NTILES
_CHUNK = 125
_NCHUNK = _EPT // _CHUNK
_SB = 32
_NPAD = 10240
_ZROWS = _NPAD // _NTILES

def _degree_body(ei_hbm, zeros_hbm, ones_hbm, deg_hbm, idx_v, ones_v, acc):
    c = lax.axis_index("c")
    s = lax.axis_index("s")
    pltpu.sync_copy(zeros_hbm.at[pl.ds(0, _ZROWS)], acc.at[pl.ds(s * _ZROWS, _ZROWS)])
    pltpu.sync_copy(ones_hbm, ones_v)
    pltpu.sync_copy(ei_hbm.at[c, s], idx_v)
    plsc.subcore_barrier()

    def body(j, carry):
        pltpu.sync_copy(ones_v, acc.at[idx_v.at[j]], add=True)
        return carry

    lax.fori_loop(0, _NCHUNK, body, 0)
    plsc.subcore_barrier()
    pltpu.sync_copy(acc.at[pl.ds(s * _ZROWS, _ZROWS)],
                    deg_hbm.at[c, pl.ds(s * _ZROWS, _ZROWS)])


def _proj_body(x_ref, wsrc_ref, wdst_ref, deg_ref, zsrc_ref, zdst_ref):
    i = pl.program_id(0)
    bm = x_ref.shape[0]
    xb = x_ref[...]
    dout = deg_ref[0, pl.ds(i * bm, bm)]
    din = deg_ref[1, pl.ds(i * bm, bm)]
    so = jnp.where(dout > 0, lax.rsqrt(dout), 0.0)
    si = jnp.where(din > 0, lax.rsqrt(din), 0.0)
    dims = (((1,), (1,)), ((), ()))
    ysrc = lax.dot_general(xb, wsrc_ref[...], dims, preferred_element_type=jnp.float32)
    ydst = lax.dot_general(xb, wdst_ref[...], dims, preferred_element_type=jnp.float32)
    zsrc_ref[...] = (_ALPHA * ysrc) * si[:, None]
    zdst_ref[...] = ((1.0 - _ALPHA) * ydst) * so[:, None]


def _project(x, w_src, w_dst, deg):
    bm = 2048
    grid = (pl.cdiv(_N, bm),)
    return pl.pallas_call(
        _proj_body,
        grid=grid,
        in_specs=[
            pl.BlockSpec((bm, _D), lambda i: (i, 0)),
            pl.BlockSpec((_D, _D), lambda i: (0, 0)),
            pl.BlockSpec((_D, _D), lambda i: (0, 0)),
            pl.BlockSpec((2, _NPAD), lambda i: (0, 0)),
        ],
        out_specs=[
            pl.BlockSpec((bm, _D), lambda i: (i, 0)),
            pl.BlockSpec((bm, _D), lambda i: (i, 0)),
        ],
        out_shape=[
            jax.ShapeDtypeStruct((_N, _D), jnp.float32),
            jax.ShapeDtypeStruct((_N, _D), jnp.float32),
        ],
    )(x, w_src, w_dst, deg)


def _agg_body(ei_hbm, zsrc_hbm, zdst_hbm, zeros_hbm, sf_hbm, sb_hbm,
              gidx_v, sidx_v, rows_v, acc, sem0, sem1):
    c = lax.axis_index("c")
    s = lax.axis_index("s")

    def run(table_hbm, out_hbm, gdir, sdir):
        pltpu.sync_copy(zeros_hbm, acc.at[pl.ds(s * _ZROWS, _ZROWS), :])
        plsc.subcore_barrier()

        def sb_body(b, carry):
            pltpu.sync_copy(ei_hbm.at[gdir, s, pl.ds(b * _SB, _SB)], gidx_v)
            pltpu.sync_copy(ei_hbm.at[sdir, s, pl.ds(b * _SB, _SB)], sidx_v)
            pltpu.async_copy(table_hbm.at[gidx_v.at[0]], rows_v.at[0], sem0)

            def body(j, carry2):
                cur = lax.rem(j, 2)
                nxt = lax.rem(j + 1, 2)
                @pl.when(cur == 0)
                def _():
                    pltpu.make_async_copy(table_hbm.at[gidx_v.at[j]], rows_v.at[0], sem0).wait()

                @pl.when(cur == 1)
                def _():
                    pltpu.make_async_copy(table_hbm.at[gidx_v.at[j]], rows_v.at[1], sem1).wait()

                @pl.when(jnp.logical_and(j + 1 < _SB, nxt == 0))
                def _():
                    pltpu.async_copy(table_hbm.at[gidx_v.at[j + 1]], rows_v.at[0], sem0)

                @pl.when(jnp.logical_and(j + 1 < _SB, nxt == 1))
                def _():
                    pltpu.async_copy(table_hbm.at[gidx_v.at[j + 1]], rows_v.at[1], sem1)

                @pl.when(cur == 0)
                def _():
                    pltpu.sync_copy(rows_v.at[0], acc.at[sidx_v.at[j]], add=True)

                @pl.when(cur == 1)
                def _():
                    pltpu.sync_copy(rows_v.at[1], acc.at[sidx_v.at[j]], add=True)

                return carry2

            lax.fori_loop(0, _SB, body, 0)
            return carry

        lax.fori_loop(0, _NCHUNK // _SB, sb_body, 0)
        plsc.subcore_barrier()
        pltpu.sync_copy(acc.at[pl.ds(s * _ZROWS, _ZROWS), :],
                        out_hbm.at[pl.ds(s * _ZROWS, _ZROWS), :])

    @pl.when(c == 0)
    def _():
        run(zsrc_hbm, sf_hbm, 1, 0)

    @pl.when(c == 1)
    def _():
        run(zdst_hbm, sb_hbm, 0, 1)


def _epi_body(sf_ref, sb_ref, deg_ref, bsrc_ref, bdst_ref, o_ref):
    i = pl.program_id(0)
    bm = sf_ref.shape[0]
    dout = deg_ref[0, pl.ds(i * bm, bm)]
    din = deg_ref[1, pl.ds(i * bm, bm)]
    so = jnp.where(dout > 0, lax.rsqrt(dout), 0.0)
    si = jnp.where(din > 0, lax.rsqrt(din), 0.0)
    bias = _ALPHA * bsrc_ref[...] + (1.0 - _ALPHA) * bdst_ref[...]
    o_ref[...] = sf_ref[...] * so[:, None] + sb_ref[...] * si[:, None] + bias


def _epilogue(sf, sb, deg, b_src, b_dst):
    bm = 2048
    grid = (pl.cdiv(_N, bm),)
    return pl.pallas_call(
        _epi_body,
        grid=grid,
        in_specs=[
            pl.BlockSpec((bm, _D), lambda i: (i, 0)),
            pl.BlockSpec((bm, _D), lambda i: (i, 0)),
            pl.BlockSpec((2, _NPAD), lambda i: (0, 0)),
            pl.BlockSpec((1, _D), lambda i: (0, 0)),
            pl.BlockSpec((1, _D), lambda i: (0, 0)),
        ],
        out_specs=pl.BlockSpec((bm, _D), lambda i: (i, 0)),
        out_shape=jax.ShapeDtypeStruct((_N, _D), jnp.float32),
    )(sf, sb, deg, b_src, b_dst)


@functools.cache
def _sc_kernels():
    mesh = plsc.VectorSubcoreMesh(core_axis_name="c", subcore_axis_name="s")
    degree_kernel = pl.kernel(
        _degree_body,
        mesh=mesh,
        out_type=jax.ShapeDtypeStruct((2, _NPAD), jnp.float32),
        scratch_types=[
            pltpu.VMEM((_NCHUNK, _CHUNK), jnp.int32),
            pltpu.VMEM((_CHUNK,), jnp.float32),
            pltpu.VMEM_SHARED((_NPAD,), jnp.float32),
        ],
    )
    agg_kernel = pl.kernel(
        _agg_body,
        mesh=mesh,
        out_type=[
            jax.ShapeDtypeStruct((_NPAD, _D), jnp.float32),
            jax.ShapeDtypeStruct((_NPAD, _D), jnp.float32),
        ],
        scratch_types=[
            pltpu.VMEM((_SB, _CHUNK), jnp.int32),
            pltpu.VMEM((_SB, _CHUNK), jnp.int32),
            pltpu.VMEM((2, _CHUNK, _D), jnp.float32),
            pltpu.VMEM_SHARED((_NPAD, _D), jnp.float32),
            pltpu.SemaphoreType.DMA,
            pltpu.SemaphoreType.DMA,
        ],
    )
    return degree_kernel, agg_kernel


def kernel(x, edge_index, W_src, b_src, W_dst, b_dst):
    degree_kernel, agg_kernel = _sc_kernels()
    ei = edge_index.astype(jnp.int32).reshape(2, _NTILES, _NCHUNK, _CHUNK)
    zeros1 = jnp.zeros((_NPAD,), jnp.float32)
    ones = jnp.ones((_CHUNK,), jnp.float32)
    zeros2 = jnp.zeros((_ZROWS, _D), jnp.float32)

    deg = degree_kernel(ei, zeros1, ones)
    z_src, z_dst = _project(x, W_src, W_dst, deg)
    s_fwd, s_bwd = agg_kernel(ei, z_src, z_dst, zeros2)
    return _epilogue(s_fwd, s_bwd, deg,
                     b_src.reshape(1, _D), b_dst.reshape(1, _D))
